# baseline (speedup 1.0000x reference)
import functools

import jax
import jax.numpy as jnp
from jax import lax
from jax.experimental import pallas as pl
from jax.experimental.pallas import tpu as pltpu
from jax.experimental.pallas import tpu_sc as plsc

NC = 2
NS = 16
NW = NC * NS
L = 16

CH = 128


def _sc_spmm(seq, src, dst, w):
    n_nodes, n_ft = seq.shape
    n_edges = src.shape[0]
    n_chunks = n_edges // CH
    assert n_chunks * CH == n_edges
    rows_per_tile = (n_nodes // (8 * NS)) * 8
    tail = n_nodes - rows_per_tile * NS
    assert tail % 8 == 0 and tail <= CH
    fg = n_ft // L

    mesh = plsc.VectorSubcoreMesh(
        core_axis_name="c", subcore_axis_name="s", num_cores=NC, num_subcores=NS
    )

    per_tile = n_chunks // NW
    n_rem = n_chunks % NW
    NB = 3
    assert per_tile % NB == 0 and per_tile >= 2 * NB

    @functools.partial(
        pl.kernel,
        out_type=jax.ShapeDtypeStruct((NC, n_nodes, n_ft), jnp.float32),
        mesh=mesh,
        scratch_types=[
            pltpu.VMEM((NB, CH), jnp.int32),
            pltpu.VMEM((NB, CH), jnp.int32),
            pltpu.VMEM((NB, CH), jnp.float32),
            pltpu.VMEM((NB, CH, n_ft), jnp.float32),
            pltpu.VMEM_SHARED((n_nodes, n_ft), jnp.float32),
        ] + [pltpu.SemaphoreType.DMA] * (4 * NB),
    )
    def k(seq_hbm, src_hbm, dst_hbm, w_hbm, out_hbm, src_r, dst_r, w_r,
          rows_r, acc_sh, *sems):
        isem = sems[0:NB]
        gsem = sems[NB:2 * NB]
        ssem = sems[2 * NB:3 * NB]
        dsem = sems[3 * NB:4 * NB]
        c = lax.axis_index("c")
        s = lax.axis_index("s")
        wid = s * NC + c

        def zero_row(i, _):
            for j in range(fg):
                rows_r[0, i, pl.ds(j * L, L)] = jnp.zeros((L,), jnp.float32)
            return 0
        lax.fori_loop(0, CH, zero_row, 0)
        zbuf = rows_r.at[0]
        base = s * rows_per_tile
        nz = rows_per_tile // CH
        for kk in range(nz):
            pltpu.sync_copy(zbuf, acc_sh.at[pl.ds(base + kk * CH, CH)])
        rem = rows_per_tile - nz * CH
        if rem:
            pltpu.sync_copy(zbuf.at[pl.ds(0, rem)],
                            acc_sh.at[pl.ds(base + nz * CH, rem)])
        if tail:
            @pl.when(s == 0)
            def _():
                pltpu.sync_copy(
                    zbuf.at[pl.ds(0, tail)],
                    acc_sh.at[pl.ds(NS * rows_per_tile, tail)])
        plsc.subcore_barrier()

        def stage_src_w(b, j):
            ebase = (wid + j * NW) * CH
            pltpu.async_copy(src_hbm.at[pl.ds(ebase, CH)], src_r.at[b], isem[b])
            pltpu.async_copy(w_hbm.at[pl.ds(ebase, CH)], w_r.at[b], isem[b])

        def stage_dst(b, j):
            ebase = (wid + j * NW) * CH
            pltpu.async_copy(dst_hbm.at[pl.ds(ebase, CH)], dst_r.at[b], dsem[b])

        def launch_gather(b, j):
            ebase = (wid + j * NW) * CH
            pltpu.make_async_copy(
                src_hbm.at[pl.ds(ebase, CH)], src_r.at[b], isem[b]).wait()
            pltpu.make_async_copy(
                w_hbm.at[pl.ds(ebase, CH)], w_r.at[b], isem[b]).wait()
            pltpu.async_copy(seq_hbm.at[src_r.at[b]], rows_r.at[b], gsem[b])

        def wait_gather(b):
            pltpu.make_async_copy(
                seq_hbm.at[src_r.at[b]], rows_r.at[b], gsem[b]).wait()

        def scatter(b, j):
            ebase = (wid + j * NW) * CH
            pltpu.make_async_copy(
                dst_hbm.at[pl.ds(ebase, CH)], dst_r.at[b], dsem[b]).wait()
            pltpu.async_copy(rows_r.at[b], acc_sh.at[dst_r.at[b]], ssem[b],
                             add=True)

        def wait_scatter(b):
            pltpu.make_async_copy(
                rows_r.at[b], acc_sh.at[dst_r.at[b]], ssem[b]).wait()

        def scale(b):
            def grp(g, _):
                w16 = w_r[b, pl.ds(g * L, L)]
                for i in range(L):
                    e = g * L + i
                    idx = jnp.full((L, 1), i, jnp.int32)
                    wb = lax.gather(
                        w16, idx,
                        lax.GatherDimensionNumbers(
                            offset_dims=(), collapsed_slice_dims=(0,),
                            start_index_map=(0,)),
                        slice_sizes=(1,),
                        mode=lax.GatherScatterMode.PROMISE_IN_BOUNDS)
                    for j in range(fg):
                        rows_r[b, e, pl.ds(j * L, L)] = (
                            rows_r[b, e, pl.ds(j * L, L)] * wb)
                return 0
            lax.fori_loop(0, CH // L, grp, 0)

        D = NB - 1

        def step(b, j, *, wait_prev=True, prefetch=True):
            bD = (b + D) % NB
            if prefetch:
                stage_src_w(bD, j + D)
            wait_gather(b)
            scale(b)
            scatter(b, j)
            if wait_prev:
                wait_scatter(bD)
            if prefetch:
                stage_dst(bD, j + D)
                launch_gather(bD, j + D)

        for j0 in range(D):
            stage_src_w(j0, j0)
            stage_dst(j0, j0)
        for j0 in range(D):
            launch_gather(j0, j0)
        for j0 in range(NB):
            step(j0, j0, wait_prev=(j0 >= NB - D))

        def mid(t, _):
            j = NB * t
            for b in range(NB):
                step(b, j + b)
            return 0
        lax.fori_loop(1, per_tile // NB - 1, mid, 0)

        j_last = per_tile - NB
        for m in range(NB):
            step(m, j_last + m, prefetch=(j_last + m + D < per_tile))
        for m in range(NB - D):
            wait_scatter((per_tile - (NB - D) + m) % NB)

        if n_rem:
            @pl.when(wid < n_rem)
            def _():
                stage_src_w(0, per_tile)
                stage_dst(0, per_tile)
                launch_gather(0, per_tile)
                wait_gather(0)
                scale(0)
                scatter(0, per_tile)
                wait_scatter(0)

        plsc.subcore_barrier()
        pltpu.sync_copy(acc_sh.at[pl.ds(base, rows_per_tile)],
                        out_hbm.at[c].at[pl.ds(base, rows_per_tile)])
        if tail:
            @pl.when(s == 0)
            def _():
                pltpu.sync_copy(
                    acc_sh.at[pl.ds(NS * rows_per_tile, tail)],
                    out_hbm.at[c].at[pl.ds(NS * rows_per_tile, tail)])

    return k(seq, src, dst, w)


def _tc_finish(partials, W, a):
    nc, n_nodes, n_ft = partials.shape
    blk = 1000
    grid = n_nodes // blk
    assert grid * blk == n_nodes

    def body(p_ref, w_ref, a_ref, o_ref):
        ps = p_ref[0] + p_ref[1]
        y = lax.dot_general(ps, w_ref[...], (((1,), (1,)), ((), ())),
                            preferred_element_type=jnp.float32)
        av = a_ref[0, 0]
        o_ref[...] = jnp.where(y >= 0, y, av * y)

    return pl.pallas_call(
        body,
        grid=(grid,),
        in_specs=[
            pl.BlockSpec((nc, blk, n_ft), lambda i: (0, i, 0)),
            pl.BlockSpec((n_ft, n_ft), lambda i: (0, 0)),
            pl.BlockSpec((1, 1), lambda i: (0, 0)),
        ],
        out_specs=pl.BlockSpec((blk, n_ft), lambda i: (i, 0)),
        out_shape=jax.ShapeDtypeStruct((n_nodes, n_ft), jnp.float32),
    )(partials, W, a.reshape(1, 1))


def kernel(seq, edge_index, edge_weight, W, a):
    ei = edge_index.astype(jnp.int32)
    dst = ei[0]
    src = ei[1]
    partials = _sc_spmm(seq, src, dst, edge_weight)
    return _tc_finish(partials, W, a)


# speedup vs baseline: 1.0295x; 1.0295x over previous
import functools

import jax
import jax.numpy as jnp
from jax import lax
from jax.experimental import pallas as pl
from jax.experimental.pallas import tpu as pltpu
from jax.experimental.pallas import tpu_sc as plsc

NC = 2
NS = 16
NW = NC * NS
L = 16

CH = 64


def _sc_spmm(seq, src, dst, w):
    n_nodes, n_ft = seq.shape
    n_edges = src.shape[0]
    n_chunks = n_edges // CH
    assert n_chunks * CH == n_edges
    rows_per_tile = (n_nodes // (8 * NS)) * 8
    tail = n_nodes - rows_per_tile * NS
    assert tail % 8 == 0 and tail <= CH
    fg = n_ft // L

    mesh = plsc.VectorSubcoreMesh(
        core_axis_name="c", subcore_axis_name="s", num_cores=NC, num_subcores=NS
    )

    per_tile = n_chunks // NW
    n_rem = n_chunks % NW
    NB = 4
    assert per_tile % NB == 0 and per_tile >= 2 * NB

    @functools.partial(
        pl.kernel,
        out_type=jax.ShapeDtypeStruct((NC, n_nodes, n_ft), jnp.float32),
        mesh=mesh,
        scratch_types=[
            pltpu.VMEM((NB, CH), jnp.int32),
            pltpu.VMEM((NB, CH), jnp.int32),
            pltpu.VMEM((NB, CH), jnp.float32),
            pltpu.VMEM((NB, CH, n_ft), jnp.float32),
            pltpu.VMEM_SHARED((n_nodes, n_ft), jnp.float32),
        ] + [pltpu.SemaphoreType.DMA] * (4 * NB),
    )
    def k(seq_hbm, src_hbm, dst_hbm, w_hbm, out_hbm, src_r, dst_r, w_r,
          rows_r, acc_sh, *sems):
        isem = sems[0:NB]
        gsem = sems[NB:2 * NB]
        ssem = sems[2 * NB:3 * NB]
        dsem = sems[3 * NB:4 * NB]
        c = lax.axis_index("c")
        s = lax.axis_index("s")
        wid = s * NC + c

        def zero_row(i, _):
            for j in range(fg):
                rows_r[0, i, pl.ds(j * L, L)] = jnp.zeros((L,), jnp.float32)
            return 0
        lax.fori_loop(0, CH, zero_row, 0)
        zbuf = rows_r.at[0]
        base = s * rows_per_tile
        nz = rows_per_tile // CH
        for kk in range(nz):
            pltpu.sync_copy(zbuf, acc_sh.at[pl.ds(base + kk * CH, CH)])
        rem = rows_per_tile - nz * CH
        if rem:
            pltpu.sync_copy(zbuf.at[pl.ds(0, rem)],
                            acc_sh.at[pl.ds(base + nz * CH, rem)])
        if tail:
            @pl.when(s == 0)
            def _():
                pltpu.sync_copy(
                    zbuf.at[pl.ds(0, tail)],
                    acc_sh.at[pl.ds(NS * rows_per_tile, tail)])
        plsc.subcore_barrier()

        def stage_src_w(b, j):
            ebase = (wid + j * NW) * CH
            pltpu.async_copy(src_hbm.at[pl.ds(ebase, CH)], src_r.at[b], isem[b])
            pltpu.async_copy(w_hbm.at[pl.ds(ebase, CH)], w_r.at[b], isem[b])

        def stage_dst(b, j):
            ebase = (wid + j * NW) * CH
            pltpu.async_copy(dst_hbm.at[pl.ds(ebase, CH)], dst_r.at[b], dsem[b])

        def launch_gather(b, j):
            ebase = (wid + j * NW) * CH
            pltpu.make_async_copy(
                src_hbm.at[pl.ds(ebase, CH)], src_r.at[b], isem[b]).wait()
            pltpu.make_async_copy(
                w_hbm.at[pl.ds(ebase, CH)], w_r.at[b], isem[b]).wait()
            pltpu.async_copy(seq_hbm.at[src_r.at[b]], rows_r.at[b], gsem[b])

        def wait_gather(b):
            pltpu.make_async_copy(
                seq_hbm.at[src_r.at[b]], rows_r.at[b], gsem[b]).wait()

        def scatter(b, j):
            ebase = (wid + j * NW) * CH
            pltpu.make_async_copy(
                dst_hbm.at[pl.ds(ebase, CH)], dst_r.at[b], dsem[b]).wait()
            pltpu.async_copy(rows_r.at[b], acc_sh.at[dst_r.at[b]], ssem[b],
                             add=True)

        def wait_scatter(b):
            pltpu.make_async_copy(
                rows_r.at[b], acc_sh.at[dst_r.at[b]], ssem[b]).wait()

        def scale(b):
            def grp(g, _):
                w16 = w_r[b, pl.ds(g * L, L)]
                for i in range(L):
                    e = g * L + i
                    idx = jnp.full((L, 1), i, jnp.int32)
                    wb = lax.gather(
                        w16, idx,
                        lax.GatherDimensionNumbers(
                            offset_dims=(), collapsed_slice_dims=(0,),
                            start_index_map=(0,)),
                        slice_sizes=(1,),
                        mode=lax.GatherScatterMode.PROMISE_IN_BOUNDS)
                    for j in range(fg):
                        rows_r[b, e, pl.ds(j * L, L)] = (
                            rows_r[b, e, pl.ds(j * L, L)] * wb)
                return 0
            lax.fori_loop(0, CH // L, grp, 0)

        D = NB - 1

        def step(b, j, *, wait_prev=True, prefetch=True):
            bD = (b + D) % NB
            if prefetch:
                stage_src_w(bD, j + D)
            wait_gather(b)
            scale(b)
            scatter(b, j)
            if wait_prev:
                wait_scatter(bD)
            if prefetch:
                stage_dst(bD, j + D)
                launch_gather(bD, j + D)

        for j0 in range(D):
            stage_src_w(j0, j0)
            stage_dst(j0, j0)
        for j0 in range(D):
            launch_gather(j0, j0)
        for j0 in range(NB):
            step(j0, j0, wait_prev=(j0 >= NB - D))

        def mid(t, _):
            j = NB * t
            for b in range(NB):
                step(b, j + b)
            return 0
        lax.fori_loop(1, per_tile // NB - 1, mid, 0)

        j_last = per_tile - NB
        for m in range(NB):
            step(m, j_last + m, prefetch=(j_last + m + D < per_tile))
        for m in range(NB - D):
            wait_scatter((per_tile - (NB - D) + m) % NB)

        if n_rem:
            @pl.when(wid < n_rem)
            def _():
                stage_src_w(0, per_tile)
                stage_dst(0, per_tile)
                launch_gather(0, per_tile)
                wait_gather(0)
                scale(0)
                scatter(0, per_tile)
                wait_scatter(0)

        plsc.subcore_barrier()
        pltpu.sync_copy(acc_sh.at[pl.ds(base, rows_per_tile)],
                        out_hbm.at[c].at[pl.ds(base, rows_per_tile)])
        if tail:
            @pl.when(s == 0)
            def _():
                pltpu.sync_copy(
                    acc_sh.at[pl.ds(NS * rows_per_tile, tail)],
                    out_hbm.at[c].at[pl.ds(NS * rows_per_tile, tail)])

    return k(seq, src, dst, w)


def _tc_finish(partials, W, a):
    nc, n_nodes, n_ft = partials.shape
    blk = 1000
    grid = n_nodes // blk
    assert grid * blk == n_nodes

    def body(p_ref, w_ref, a_ref, o_ref):
        ps = p_ref[0] + p_ref[1]
        y = lax.dot_general(ps, w_ref[...], (((1,), (1,)), ((), ())),
                            preferred_element_type=jnp.float32)
        av = a_ref[0, 0]
        o_ref[...] = jnp.where(y >= 0, y, av * y)

    return pl.pallas_call(
        body,
        grid=(grid,),
        in_specs=[
            pl.BlockSpec((nc, blk, n_ft), lambda i: (0, i, 0)),
            pl.BlockSpec((n_ft, n_ft), lambda i: (0, 0)),
            pl.BlockSpec((1, 1), lambda i: (0, 0)),
        ],
        out_specs=pl.BlockSpec((blk, n_ft), lambda i: (i, 0)),
        out_shape=jax.ShapeDtypeStruct((n_nodes, n_ft), jnp.float32),
    )(partials, W, a.reshape(1, 1))


def kernel(seq, edge_index, edge_weight, W, a):
    ei = edge_index.astype(jnp.int32)
    dst = ei[0]
    src = ei[1]
    partials = _sc_spmm(seq, src, dst, edge_weight)
    return _tc_finish(partials, W, a)


# speedup vs baseline: 1.0482x; 1.0182x over previous
import functools

import jax
import jax.numpy as jnp
from jax import lax
from jax.experimental import pallas as pl
from jax.experimental.pallas import tpu as pltpu
from jax.experimental.pallas import tpu_sc as plsc

NC = 2
NS = 16
NW = NC * NS
L = 16

CH = 64


def _sc_spmm(seq, src, dst, w):
    n_nodes, n_ft = seq.shape
    n_edges = src.shape[0]
    n_chunks = n_edges // CH
    assert n_chunks * CH == n_edges
    rows_per_tile = (n_nodes // (8 * NS)) * 8
    tail = n_nodes - rows_per_tile * NS
    assert tail % 8 == 0 and tail <= CH
    fg = n_ft // L

    mesh = plsc.VectorSubcoreMesh(
        core_axis_name="c", subcore_axis_name="s", num_cores=NC, num_subcores=NS
    )

    per_tile = n_chunks // NW
    n_rem = n_chunks % NW
    NB = 4
    assert per_tile % NB == 0 and per_tile >= 2 * NB

    @functools.partial(
        pl.kernel,
        out_type=jax.ShapeDtypeStruct((NC, n_nodes, n_ft), jnp.float32),
        mesh=mesh,
        scratch_types=[
            pltpu.VMEM((NB, CH), jnp.int32),
            pltpu.VMEM((NB, CH), jnp.int32),
            pltpu.VMEM((NB, CH), jnp.float32),
            pltpu.VMEM((NB, CH, n_ft), jnp.float32),
            pltpu.VMEM_SHARED((n_nodes, n_ft), jnp.float32),
        ] + [pltpu.SemaphoreType.DMA] * (4 * NB),
    )
    def k(seq_hbm, src_hbm, dst_hbm, w_hbm, out_hbm, src_r, dst_r, w_r,
          rows_r, acc_sh, *sems):
        isem = sems[0:NB]
        gsem = sems[NB:2 * NB]
        ssem = sems[2 * NB:3 * NB]
        dsem = sems[3 * NB:4 * NB]
        c = lax.axis_index("c")
        s = lax.axis_index("s")
        wid = s * NC + c

        def zero_row(i, _):
            for j in range(fg):
                rows_r[0, i, pl.ds(j * L, L)] = jnp.zeros((L,), jnp.float32)
            return 0
        lax.fori_loop(0, CH, zero_row, 0)
        zbuf = rows_r.at[0]
        base = s * rows_per_tile
        nz = rows_per_tile // CH
        for kk in range(nz):
            pltpu.sync_copy(zbuf, acc_sh.at[pl.ds(base + kk * CH, CH)])
        rem = rows_per_tile - nz * CH
        if rem:
            pltpu.sync_copy(zbuf.at[pl.ds(0, rem)],
                            acc_sh.at[pl.ds(base + nz * CH, rem)])
        if tail:
            @pl.when(s == 0)
            def _():
                pltpu.sync_copy(
                    zbuf.at[pl.ds(0, tail)],
                    acc_sh.at[pl.ds(NS * rows_per_tile, tail)])
        plsc.subcore_barrier()

        def stage_src_w(b, j):
            ebase = (wid + j * NW) * CH
            pltpu.async_copy(src_hbm.at[pl.ds(ebase, CH)], src_r.at[b], isem[b])
            pltpu.async_copy(w_hbm.at[pl.ds(ebase, CH)], w_r.at[b], isem[b])

        def stage_dst(b, j):
            ebase = (wid + j * NW) * CH
            pltpu.async_copy(dst_hbm.at[pl.ds(ebase, CH)], dst_r.at[b], dsem[b])

        def launch_gather(b, j):
            ebase = (wid + j * NW) * CH
            pltpu.make_async_copy(
                src_hbm.at[pl.ds(ebase, CH)], src_r.at[b], isem[b]).wait()
            pltpu.make_async_copy(
                w_hbm.at[pl.ds(ebase, CH)], w_r.at[b], isem[b]).wait()
            pltpu.async_copy(seq_hbm.at[src_r.at[b]], rows_r.at[b], gsem[b])

        def wait_gather(b):
            pltpu.make_async_copy(
                seq_hbm.at[src_r.at[b]], rows_r.at[b], gsem[b]).wait()

        def scatter(b, j):
            ebase = (wid + j * NW) * CH
            pltpu.make_async_copy(
                dst_hbm.at[pl.ds(ebase, CH)], dst_r.at[b], dsem[b]).wait()
            pltpu.async_copy(rows_r.at[b], acc_sh.at[dst_r.at[b]], ssem[b],
                             add=True)

        def wait_scatter(b):
            pltpu.make_async_copy(
                rows_r.at[b], acc_sh.at[dst_r.at[b]], ssem[b]).wait()

        def scale(b):
            def grp(g, _):
                w16 = w_r[b, pl.ds(g * L, L)]
                for i in range(L):
                    e = g * L + i
                    idx = jnp.full((L, 1), i, jnp.int32)
                    wb = lax.gather(
                        w16, idx,
                        lax.GatherDimensionNumbers(
                            offset_dims=(), collapsed_slice_dims=(0,),
                            start_index_map=(0,)),
                        slice_sizes=(1,),
                        mode=lax.GatherScatterMode.PROMISE_IN_BOUNDS)
                    for j in range(fg):
                        rows_r[b, e, pl.ds(j * L, L)] = (
                            rows_r[b, e, pl.ds(j * L, L)] * wb)
                return 0
            lax.fori_loop(0, CH // L, grp, 0)

        D = NB - 1

        def step(b, j, *, wait_prev=True, prefetch=True):
            bD = (b + D) % NB
            if prefetch:
                stage_src_w(bD, j + D)
            wait_gather(b)
            scale(b)
            scatter(b, j)
            if wait_prev:
                wait_scatter(bD)
            if prefetch:
                stage_dst(bD, j + D)
                launch_gather(bD, j + D)

        for j0 in range(D):
            stage_src_w(j0, j0)
            stage_dst(j0, j0)
        for j0 in range(D):
            launch_gather(j0, j0)
        for j0 in range(NB):
            step(j0, j0, wait_prev=(j0 >= NB - D))

        def mid(t, _):
            j = NB * t
            for b in range(NB):
                step(b, j + b)
            return 0
        lax.fori_loop(1, per_tile // NB - 1, mid, 0)

        j_last = per_tile - NB
        for m in range(NB):
            step(m, j_last + m, prefetch=(j_last + m + D < per_tile))
        for m in range(NB - D):
            wait_scatter((per_tile - (NB - D) + m) % NB)

        if n_rem:
            @pl.when(wid < n_rem)
            def _():
                stage_src_w(0, per_tile)
                stage_dst(0, per_tile)
                launch_gather(0, per_tile)
                wait_gather(0)
                scale(0)
                scatter(0, per_tile)
                wait_scatter(0)

        plsc.subcore_barrier()
        pltpu.sync_copy(acc_sh.at[pl.ds(base, rows_per_tile)],
                        out_hbm.at[c].at[pl.ds(base, rows_per_tile)])
        if tail:
            @pl.when(s == 0)
            def _():
                pltpu.sync_copy(
                    acc_sh.at[pl.ds(NS * rows_per_tile, tail)],
                    out_hbm.at[c].at[pl.ds(NS * rows_per_tile, tail)])

    return k(seq, src, dst, w)


def _tc_finish(partials, W, a):
    nc, n_nodes, n_ft = partials.shape
    blk = 2000
    grid = n_nodes // blk
    assert grid * blk == n_nodes

    def body(p_ref, w_ref, a_ref, o_ref):
        ps = p_ref[0] + p_ref[1]
        y = lax.dot_general(ps, w_ref[...], (((1,), (1,)), ((), ())),
                            preferred_element_type=jnp.float32)
        av = a_ref[0, 0]
        o_ref[...] = jnp.where(y >= 0, y, av * y)

    return pl.pallas_call(
        body,
        grid=(grid,),
        in_specs=[
            pl.BlockSpec((nc, blk, n_ft), lambda i: (0, i, 0)),
            pl.BlockSpec((n_ft, n_ft), lambda i: (0, 0)),
            pl.BlockSpec((1, 1), lambda i: (0, 0)),
        ],
        out_specs=pl.BlockSpec((blk, n_ft), lambda i: (i, 0)),
        out_shape=jax.ShapeDtypeStruct((n_nodes, n_ft), jnp.float32),
    )(partials, W, a.reshape(1, 1))


def kernel(seq, edge_index, edge_weight, W, a):
    ei = edge_index.astype(jnp.int32)
    dst = ei[0]
    src = ei[1]
    partials = _sc_spmm(seq, src, dst, edge_weight)
    return _tc_finish(partials, W, a)


# speedup vs baseline: 1.0613x; 1.0125x over previous
import functools

import jax
import jax.numpy as jnp
from jax import lax
from jax.experimental import pallas as pl
from jax.experimental.pallas import tpu as pltpu
from jax.experimental.pallas import tpu_sc as plsc

NC = 2
NS = 16
NW = NC * NS
L = 16

CH = 64


def _sc_spmm(seq, src, dst, w):
    n_nodes, n_ft = seq.shape
    n_edges = src.shape[0]
    n_chunks = n_edges // CH
    assert n_chunks * CH == n_edges
    rows_per_tile = (n_nodes // (8 * NS)) * 8
    tail = n_nodes - rows_per_tile * NS
    assert tail % 8 == 0 and tail <= CH
    fg = n_ft // L

    mesh = plsc.VectorSubcoreMesh(
        core_axis_name="c", subcore_axis_name="s", num_cores=NC, num_subcores=NS
    )

    per_tile = n_chunks // NW
    n_rem = n_chunks % NW
    NB = 4
    assert per_tile % NB == 0 and per_tile >= 2 * NB

    @functools.partial(
        pl.kernel,
        out_type=jax.ShapeDtypeStruct((NC, n_nodes, n_ft), jnp.float32),
        mesh=mesh,
        scratch_types=[
            pltpu.VMEM((NB, CH), jnp.int32),
            pltpu.VMEM((NB, CH), jnp.int32),
            pltpu.VMEM((NB, CH), jnp.float32),
            pltpu.VMEM((NB, CH, n_ft), jnp.float32),
            pltpu.VMEM_SHARED((n_nodes, n_ft), jnp.float32),
        ] + [pltpu.SemaphoreType.DMA] * (4 * NB),
    )
    def k(seq_hbm, src_hbm, dst_hbm, w_hbm, out_hbm, src_r, dst_r, w_r,
          rows_r, acc_sh, *sems):
        isem = sems[0:NB]
        gsem = sems[NB:2 * NB]
        ssem = sems[2 * NB:3 * NB]
        dsem = sems[3 * NB:4 * NB]
        c = lax.axis_index("c")
        s = lax.axis_index("s")
        wid = s * NC + c

        def zero_row(i, _):
            for j in range(fg):
                rows_r[0, i, pl.ds(j * L, L)] = jnp.zeros((L,), jnp.float32)
            return 0
        lax.fori_loop(0, CH, zero_row, 0)
        zbuf = rows_r.at[0]
        base = s * rows_per_tile
        nz = rows_per_tile // CH
        for kk in range(nz):
            pltpu.sync_copy(zbuf, acc_sh.at[pl.ds(base + kk * CH, CH)])
        rem = rows_per_tile - nz * CH
        if rem:
            pltpu.sync_copy(zbuf.at[pl.ds(0, rem)],
                            acc_sh.at[pl.ds(base + nz * CH, rem)])
        if tail:
            @pl.when(s == 0)
            def _():
                pltpu.sync_copy(
                    zbuf.at[pl.ds(0, tail)],
                    acc_sh.at[pl.ds(NS * rows_per_tile, tail)])
        plsc.subcore_barrier()

        def stage_src_w(b, j):
            ebase = (wid + j * NW) * CH
            pltpu.async_copy(src_hbm.at[pl.ds(ebase, CH)], src_r.at[b], isem[b])
            pltpu.async_copy(w_hbm.at[pl.ds(ebase, CH)], w_r.at[b], isem[b])

        def stage_dst(b, j):
            ebase = (wid + j * NW) * CH
            pltpu.async_copy(dst_hbm.at[pl.ds(ebase, CH)], dst_r.at[b], dsem[b])

        def launch_gather(b, j):
            ebase = (wid + j * NW) * CH
            pltpu.make_async_copy(
                src_hbm.at[pl.ds(ebase, CH)], src_r.at[b], isem[b]).wait()
            pltpu.make_async_copy(
                w_hbm.at[pl.ds(ebase, CH)], w_r.at[b], isem[b]).wait()
            pltpu.async_copy(seq_hbm.at[src_r.at[b]], rows_r.at[b], gsem[b])

        def wait_gather(b):
            pltpu.make_async_copy(
                seq_hbm.at[src_r.at[b]], rows_r.at[b], gsem[b]).wait()

        def scatter(b, j):
            ebase = (wid + j * NW) * CH
            pltpu.make_async_copy(
                dst_hbm.at[pl.ds(ebase, CH)], dst_r.at[b], dsem[b]).wait()
            pltpu.async_copy(rows_r.at[b], acc_sh.at[dst_r.at[b]], ssem[b],
                             add=True)

        def wait_scatter(b):
            pltpu.make_async_copy(
                rows_r.at[b], acc_sh.at[dst_r.at[b]], ssem[b]).wait()

        def scale(b):
            def grp(g, _):
                w16 = w_r[b, pl.ds(g * L, L)]
                for i in range(L):
                    e = g * L + i
                    idx = jnp.full((L, 1), i, jnp.int32)
                    wb = lax.gather(
                        w16, idx,
                        lax.GatherDimensionNumbers(
                            offset_dims=(), collapsed_slice_dims=(0,),
                            start_index_map=(0,)),
                        slice_sizes=(1,),
                        mode=lax.GatherScatterMode.PROMISE_IN_BOUNDS)
                    for j in range(fg):
                        rows_r[b, e, pl.ds(j * L, L)] = (
                            rows_r[b, e, pl.ds(j * L, L)] * wb)
                return 0
            lax.fori_loop(0, CH // L, grp, 0)

        D = NB - 1

        def step(b, j, *, wait_prev=True, prefetch=True):
            bD = (b + D) % NB
            if prefetch:
                stage_src_w(bD, j + D)
            wait_gather(b)
            scale(b)
            scatter(b, j)
            if wait_prev:
                wait_scatter(bD)
            if prefetch:
                stage_dst(bD, j + D)
                launch_gather(bD, j + D)

        for j0 in range(D):
            stage_src_w(j0, j0)
            stage_dst(j0, j0)
        for j0 in range(D):
            launch_gather(j0, j0)
        for j0 in range(NB):
            step(j0, j0, wait_prev=(j0 >= NB - D))

        def mid(t, _):
            j = NB * t
            for b in range(NB):
                step(b, j + b)
            return 0
        lax.fori_loop(1, per_tile // NB - 1, mid, 0)

        j_last = per_tile - NB
        for m in range(NB):
            step(m, j_last + m, prefetch=(j_last + m + D < per_tile))
        for m in range(NB - D):
            wait_scatter((per_tile - (NB - D) + m) % NB)

        if n_rem:
            @pl.when(wid < n_rem)
            def _():
                stage_src_w(0, per_tile)
                stage_dst(0, per_tile)
                launch_gather(0, per_tile)
                wait_gather(0)
                scale(0)
                scatter(0, per_tile)
                wait_scatter(0)

        plsc.subcore_barrier()
        pltpu.sync_copy(acc_sh.at[pl.ds(base, rows_per_tile)],
                        out_hbm.at[c].at[pl.ds(base, rows_per_tile)])
        if tail:
            @pl.when(s == 0)
            def _():
                pltpu.sync_copy(
                    acc_sh.at[pl.ds(NS * rows_per_tile, tail)],
                    out_hbm.at[c].at[pl.ds(NS * rows_per_tile, tail)])

    return k(seq, src, dst, w)


def _tc_finish(partials, W, a):
    nc, n_nodes, n_ft = partials.shape
    blk = 5000
    grid = n_nodes // blk
    assert grid * blk == n_nodes

    def body(p_ref, w_ref, a_ref, o_ref):
        ps = p_ref[0] + p_ref[1]
        y = lax.dot_general(ps, w_ref[...], (((1,), (1,)), ((), ())),
                            preferred_element_type=jnp.float32)
        av = a_ref[0, 0]
        o_ref[...] = jnp.where(y >= 0, y, av * y)

    return pl.pallas_call(
        body,
        grid=(grid,),
        in_specs=[
            pl.BlockSpec((nc, blk, n_ft), lambda i: (0, i, 0)),
            pl.BlockSpec((n_ft, n_ft), lambda i: (0, 0)),
            pl.BlockSpec((1, 1), lambda i: (0, 0)),
        ],
        out_specs=pl.BlockSpec((blk, n_ft), lambda i: (i, 0)),
        out_shape=jax.ShapeDtypeStruct((n_nodes, n_ft), jnp.float32),
    )(partials, W, a.reshape(1, 1))


def kernel(seq, edge_index, edge_weight, W, a):
    ei = edge_index.astype(jnp.int32)
    dst = ei[0]
    src = ei[1]
    partials = _sc_spmm(seq, src, dst, edge_weight)
    return _tc_finish(partials, W, a)
